# baseline (speedup 1.0000x reference)
import functools

import jax
import jax.numpy as jnp
from jax import lax
from jax.experimental import pallas as pl
from jax.experimental.pallas import tpu as pltpu
from jax.experimental.pallas import tpu_sc as plsc

_H = 32
_C = 3
_S = 67
_PK_ROWS = 16
_OUT_ROWS = 72


def _pack_body(rel_ref, csb_ref, scb_ref, ccb_ref, out_ref):
    rel = rel_ref[...]
    csb = csb_ref[...]
    scb = scb_ref[...]
    ccb = ccb_ref[...]
    z1 = jnp.zeros((_H, 1), jnp.float32)
    z16 = jnp.zeros((_H, 16), jnp.float32)
    z64 = jnp.zeros((_H, 64), jnp.float32)
    z128 = jnp.zeros((_H, 128), jnp.float32)

    def pad16(x):
        return jnp.concatenate(
            [x] + [z1] * (16 - x.shape[1]), axis=-1)

    rows = []
    rows.append(jnp.concatenate([pad16(rel[:, d, :]) for d in range(8)], -1))
    rows.append(jnp.concatenate(
        [pad16(rel[:, d, :]) for d in range(8, 15)] + [z16], -1))
    rows.append(jnp.concatenate([csb[:, 0, :], csb[:, 1, :]], -1))
    rows.append(jnp.concatenate([csb[:, 2, :], z64], -1))
    for blk in range(8):
        rows.append(jnp.concatenate(
            [pad16(scb[:, 8 * blk + t, :]) for t in range(8)], -1))
    rows.append(jnp.concatenate(
        [pad16(ccb[:, r, :]) for r in range(_C)] + [z16] * 5, -1))
    rows.extend([z128] * 3)
    out_ref[...] = jnp.concatenate(
        [r[:, None, :] for r in rows], axis=1)


@jax.jit
def _pack(rel, csb, scb, ccb):
    return pl.pallas_call(
        _pack_body,
        out_shape=jax.ShapeDtypeStruct((_H, _PK_ROWS, 128), jnp.float32),
    )(rel, csb, scb, ccb)


def _i32(x):
    return jnp.full((16,), x, jnp.int32)


@functools.cache
def _gather_fn():
    mesh = plsc.VectorSubcoreMesh(core_axis_name="c", subcore_axis_name="s")

    @functools.partial(
        pl.kernel,
        out_type=jax.ShapeDtypeStruct((_H, _OUT_ROWS, 128), jnp.float32),
        mesh=mesh,
        scratch_types=[
            pltpu.VMEM((_PK_ROWS, 128), jnp.float32),
            pltpu.VMEM((_OUT_ROWS, 128), jnp.float32),
        ],
        compiler_params=pltpu.CompilerParams(
            needs_layout_passes=False, use_tc_tiling_on_sc=False),
    )
    def _gather(tbl_hbm, out_hbm, tbl_v, out_v):
        num_cores = lax.axis_size("c")
        h = lax.axis_index("s") * num_cores + lax.axis_index("c")
        pltpu.sync_copy(tbl_hbm.at[h], tbl_v)

        lane = lax.iota(jnp.int32, 16)

        def sq_row(i, carry):
            dst_r = _i32(i + _C)
            for v in range(4):
                j = lane + (16 * v)
                dr = (i >> 3) + 7 - (j >> 3)
                df = (i & 7) + 7 - (j & 7)
                src = dr * 16 + df
                vals = plsc.load_gather(tbl_v, [src >> 7, src & 127])
                plsc.store_scatter(out_v, [dst_r, j + _C], vals)
            return carry

        lax.fori_loop(0, 64, sq_row, 0)

        for r in range(_C):
            for v in range(4):
                vals = tbl_v[2 + r // 2, pl.ds(64 * (r % 2) + 16 * v, 16)]
                plsc.store_scatter(
                    out_v, [_i32(r), lane + (_C + 16 * v)], vals)

        for m in range(12):
            s = lane + (16 * m)
            q = (s * 21846) >> 16
            c = s - q * 3
            src = q * 16 + c + 512
            vals = plsc.load_gather(tbl_v, [src >> 7, src & 127])
            plsc.store_scatter(out_v, [q + _C, c], vals)

        s = lane
        q = (s * 21846) >> 16
        c = s - q * 3
        src = q * 16 + c + 1536
        m9 = s < 9
        vals = plsc.load_gather(tbl_v, [src >> 7, src & 127], mask=m9)
        plsc.store_scatter(out_v, [q, c], vals, mask=m9)

        pltpu.sync_copy(out_v, out_hbm.at[h])

    return _gather


def kernel(rel_bias, context_sq_bias, sq_context_bias, context_context_bias):
    tbl = _pack(rel_bias, context_sq_bias, sq_context_bias,
                context_context_bias)
    out = _gather_fn()(tbl)
    return out[:, :_S, :_S]


# speedup vs baseline: 1.1413x; 1.1413x over previous
import functools

import jax
import jax.numpy as jnp
from jax import lax
from jax.experimental import pallas as pl
from jax.experimental.pallas import tpu as pltpu
from jax.experimental.pallas import tpu_sc as plsc

_H = 32
_C = 3
_S = 67


def _i32(x):
    return jnp.full((16,), x, jnp.int32)


@functools.cache
def _bias_fn():
    mesh = plsc.VectorSubcoreMesh(core_axis_name="c", subcore_axis_name="s")

    @functools.partial(
        pl.kernel,
        out_type=jax.ShapeDtypeStruct((_H, _S, _S), jnp.float32),
        mesh=mesh,
        scratch_types=[
            pltpu.VMEM((15, 15), jnp.float32),
            pltpu.VMEM((_C, 64), jnp.float32),
            pltpu.VMEM((64, _C), jnp.float32),
            pltpu.VMEM((_C, _C), jnp.float32),
            pltpu.VMEM((_S, _S), jnp.float32),
            pltpu.SemaphoreType.DMA,
            pltpu.SemaphoreType.DMA,
            pltpu.SemaphoreType.DMA,
            pltpu.SemaphoreType.DMA,
        ],
        compiler_params=pltpu.CompilerParams(
            needs_layout_passes=False,
            use_tc_tiling_on_sc=False,
            disable_bounds_checks=True,
            disable_semaphore_checks=True,
            skip_device_barrier=True,
        ),
    )
    def _bias(rel_hbm, csb_hbm, scb_hbm, ccb_hbm, out_hbm,
              rel_v, csb_v, scb_v, ccb_v, out_v, sem0, sem1, sem2, sem3):
        num_cores = lax.axis_size("c")
        h = lax.axis_index("s") * num_cores + lax.axis_index("c")
        cp_rel = pltpu.make_async_copy(rel_hbm.at[h], rel_v, sem0)
        cp_csb = pltpu.make_async_copy(csb_hbm.at[h], csb_v, sem1)
        cp_scb = pltpu.make_async_copy(scb_hbm.at[h], scb_v, sem2)
        cp_ccb = pltpu.make_async_copy(ccb_hbm.at[h], ccb_v, sem3)
        cp_rel.start()
        cp_csb.start()
        cp_scb.start()
        cp_ccb.start()

        lane = lax.iota(jnp.int32, 16)

        cp_rel.wait()

        def sq_row(i, carry):
            dst_r = _i32(i + _C)
            for v in range(4):
                j = lane + (16 * v)
                dr = ((i >> 3) + 7) - (j >> 3)
                df = ((i & 7) + 7) - (j & 7)
                vals = plsc.load_gather(rel_v, [dr, df])
                plsc.store_scatter(out_v, [dst_r, j + _C], vals)
            return carry

        lax.fori_loop(0, 64, sq_row, 0)

        cp_csb.wait()
        for r in range(_C):
            for v in range(4):
                vals = csb_v[r, pl.ds(16 * v, 16)]
                plsc.store_scatter(
                    out_v, [_i32(r), lane + (_C + 16 * v)], vals)

        cp_scb.wait()
        for v in range(12):
            s = lane + (16 * v)
            q = (s * 21846) >> 16
            c = s - q * 3
            vals = plsc.load_gather(scb_v, [q, c])
            plsc.store_scatter(out_v, [q + _C, c], vals)

        cp_ccb.wait()
        s = lane
        q = (s * 21846) >> 16
        c = s - q * 3
        m = s < 9
        vals = plsc.load_gather(ccb_v, [q, c], mask=m)
        plsc.store_scatter(out_v, [q, c], vals, mask=m)

        pltpu.sync_copy(out_v, out_hbm.at[h])

    return _bias


def kernel(rel_bias, context_sq_bias, sq_context_bias, context_context_bias):
    return _bias_fn()(
        rel_bias, context_sq_bias, sq_context_bias, context_context_bias)


# speedup vs baseline: 1.1515x; 1.0089x over previous
import functools

import jax
import jax.numpy as jnp
from jax import lax
from jax.experimental import pallas as pl
from jax.experimental.pallas import tpu as pltpu
from jax.experimental.pallas import tpu_sc as plsc

_H = 32
_C = 3
_S = 67


def _i32(x):
    return jnp.full((16,), x, jnp.int32)


@functools.cache
def _bias_fn():
    mesh = plsc.VectorSubcoreMesh(core_axis_name="c", subcore_axis_name="s")

    @functools.partial(
        pl.kernel,
        out_type=jax.ShapeDtypeStruct((_H, _S, _S), jnp.float32),
        mesh=mesh,
        scratch_types=[
            pltpu.VMEM((15, 15), jnp.float32),
            pltpu.VMEM((_C, 64), jnp.float32),
            pltpu.VMEM((64, _C), jnp.float32),
            pltpu.VMEM((_C, _C), jnp.float32),
            pltpu.VMEM((_S, _S), jnp.float32),
            pltpu.SemaphoreType.DMA,
            pltpu.SemaphoreType.DMA,
            pltpu.SemaphoreType.DMA,
            pltpu.SemaphoreType.DMA,
        ],
        compiler_params=pltpu.CompilerParams(
            needs_layout_passes=False,
            use_tc_tiling_on_sc=False,
            disable_bounds_checks=True,
            disable_semaphore_checks=True,
            skip_device_barrier=True,
        ),
    )
    def _bias(rel_hbm, csb_hbm, scb_hbm, ccb_hbm, out_hbm,
              rel_v, csb_v, scb_v, ccb_v, out_v, sem0, sem1, sem2, sem3):
        num_cores = lax.axis_size("c")
        h = lax.axis_index("s") * num_cores + lax.axis_index("c")
        cp_rel = pltpu.make_async_copy(rel_hbm.at[h], rel_v, sem0)
        cp_csb = pltpu.make_async_copy(csb_hbm.at[h], csb_v, sem1)
        cp_scb = pltpu.make_async_copy(scb_hbm.at[h], scb_v, sem2)
        cp_ccb = pltpu.make_async_copy(ccb_hbm.at[h], ccb_v, sem3)
        cp_rel.start()
        cp_csb.start()
        cp_scb.start()
        cp_ccb.start()

        lane = lax.iota(jnp.int32, 16)

        cp_rel.wait()

        def sq_group(t, carry):
            i = t >> 2
            j = lane + ((t & 3) << 4)
            dr = ((i >> 3) + 7) - (j >> 3)
            df = ((i & 7) + 7) - (j & 7)
            vals = plsc.load_gather(rel_v, [dr, df])
            plsc.store_scatter(out_v, [_i32(i + _C), j + _C], vals)
            return carry

        lax.fori_loop(0, 256, sq_group, 0)

        cp_csb.wait()

        def cs_group(t, carry):
            r = t >> 2
            c = lane + ((t & 3) << 4)
            vals = plsc.load_gather(csb_v, [_i32(r), c])
            plsc.store_scatter(out_v, [_i32(r), c + _C], vals)
            return carry

        lax.fori_loop(0, 12, cs_group, 0)

        cp_scb.wait()

        def sc_group(v, carry):
            s = lane + (v << 4)
            q = (s * 21846) >> 16
            c = s - q * 3
            vals = plsc.load_gather(scb_v, [q, c])
            plsc.store_scatter(out_v, [q + _C, c], vals)
            return carry

        lax.fori_loop(0, 12, sc_group, 0)

        cp_ccb.wait()
        s = lane
        q = (s * 21846) >> 16
        c = s - q * 3
        m = s < 9
        vals = plsc.load_gather(ccb_v, [q, c], mask=m)
        plsc.store_scatter(out_v, [q, c], vals, mask=m)

        pltpu.sync_copy(out_v, out_hbm.at[h])

    return _bias


def kernel(rel_bias, context_sq_bias, sq_context_bias, context_context_bias):
    return _bias_fn()(
        rel_bias, context_sq_bias, sq_context_bias, context_context_bias)


# speedup vs baseline: 1.2298x; 1.0680x over previous
import functools

import jax
import jax.numpy as jnp
from jax import lax
from jax.experimental import pallas as pl
from jax.experimental.pallas import tpu as pltpu
from jax.experimental.pallas import tpu_sc as plsc

_H = 32
_C = 3
_S = 67


def _i32(x):
    return jnp.full((16,), x, jnp.int32)


@functools.cache
def _bias_fn():
    mesh = plsc.VectorSubcoreMesh(core_axis_name="c", subcore_axis_name="s")

    @functools.partial(
        pl.kernel,
        out_type=jax.ShapeDtypeStruct((_H, _S, _S), jnp.float32),
        mesh=mesh,
        scratch_types=[
            pltpu.VMEM((618,), jnp.float32),
            pltpu.VMEM((_S, _S), jnp.float32),
            pltpu.SemaphoreType.DMA,
        ],
        compiler_params=pltpu.CompilerParams(
            needs_layout_passes=False,
            use_tc_tiling_on_sc=False,
            disable_bounds_checks=True,
            disable_semaphore_checks=True,
            skip_device_barrier=True,
        ),
    )
    def _bias(tab_hbm, out_hbm, tab_v, out_v, sem0):
        num_cores = lax.axis_size("c")
        h = lax.axis_index("s") * num_cores + lax.axis_index("c")
        cp_tab = pltpu.make_async_copy(tab_hbm.at[h], tab_v, sem0)
        cp_tab.start()

        lane = lax.iota(jnp.int32, 16)
        cp_tab.wait()

        def sq_group(t, carry):
            i = t >> 2
            j = lane + ((t & 3) << 4)
            dr = ((i >> 3) + 7) - (j >> 3)
            df = ((i & 7) + 7) - (j & 7)
            vals = plsc.load_gather(tab_v, [dr * 15 + df])
            plsc.store_scatter(out_v, [_i32(i + _C), j + _C], vals)
            return carry

        lax.fori_loop(0, 256, sq_group, 0)

        def cs_group(t, carry):
            r = t >> 2
            c = lane + ((t & 3) << 4)
            vals = plsc.load_gather(tab_v, [(c + 225) + (r << 6)])
            plsc.store_scatter(out_v, [_i32(r), c + _C], vals)
            return carry

        lax.fori_loop(0, 12, cs_group, 0)

        def sc_group(v, carry):
            s = lane + (v << 4)
            q = (s * 21846) >> 16
            c = s - q * 3
            vals = plsc.load_gather(tab_v, [s + 417])
            plsc.store_scatter(out_v, [q + _C, c], vals)
            return carry

        lax.fori_loop(0, 12, sc_group, 0)

        s = lane
        q = (s * 21846) >> 16
        c = s - q * 3
        m = s < 9
        vals = plsc.load_gather(tab_v, [s + 609], mask=m)
        plsc.store_scatter(out_v, [q, c], vals, mask=m)

        pltpu.sync_copy(out_v, out_hbm.at[h])

    return _bias


def kernel(rel_bias, context_sq_bias, sq_context_bias, context_context_bias):
    packed = jnp.concatenate(
        [
            rel_bias.reshape(_H, 225),
            context_sq_bias.reshape(_H, 192),
            sq_context_bias.reshape(_H, 192),
            context_context_bias.reshape(_H, 9),
        ],
        axis=1,
    )
    return _bias_fn()(packed)


# speedup vs baseline: 1.2477x; 1.0145x over previous
import functools

import jax
import jax.numpy as jnp
from jax import lax
from jax.experimental import pallas as pl
from jax.experimental.pallas import tpu as pltpu
from jax.experimental.pallas import tpu_sc as plsc

_H = 32
_C = 3
_S = 67


def _i32(x):
    return jnp.full((16,), x, jnp.int32)


@functools.cache
def _bias_fn():
    mesh = plsc.VectorSubcoreMesh(core_axis_name="c", subcore_axis_name="s")

    @functools.partial(
        pl.kernel,
        out_type=jax.ShapeDtypeStruct((_H, 72, 128), jnp.float32),
        mesh=mesh,
        scratch_types=[
            pltpu.VMEM((618,), jnp.float32),
            pltpu.VMEM((72, 128), jnp.float32),
            pltpu.SemaphoreType.DMA,
        ],
        compiler_params=pltpu.CompilerParams(
            needs_layout_passes=False,
            use_tc_tiling_on_sc=False,
            disable_bounds_checks=True,
            disable_semaphore_checks=True,
            skip_device_barrier=True,
        ),
    )
    def _bias(tab_hbm, out_hbm, tab_v, out_v, sem0):
        num_cores = lax.axis_size("c")
        h = lax.axis_index("s") * num_cores + lax.axis_index("c")
        cp_tab = pltpu.make_async_copy(tab_hbm.at[h], tab_v, sem0)
        cp_tab.start()

        lane = lax.iota(jnp.int32, 16)
        cp_tab.wait()

        def sq_group(t, carry):
            i = t >> 2
            j = lane + ((t & 3) << 4)
            dr = ((i >> 3) + 7) - (j >> 3)
            df = ((i & 7) + 7) - (j & 7)
            vals = plsc.load_gather(tab_v, [dr * 15 + df])
            plsc.store_scatter(out_v, [_i32(i + _C), j + _C], vals)
            return carry

        lax.fori_loop(0, 256, sq_group, 0)

        def cs_group(t, carry):
            r = t >> 2
            c = lane + ((t & 3) << 4)
            vals = plsc.load_gather(tab_v, [(c + 225) + (r << 6)])
            plsc.store_scatter(out_v, [_i32(r), c + _C], vals)
            return carry

        lax.fori_loop(0, 12, cs_group, 0)

        def sc_group(v, carry):
            s = lane + (v << 4)
            q = (s * 21846) >> 16
            c = s - q * 3
            vals = plsc.load_gather(tab_v, [s + 417])
            plsc.store_scatter(out_v, [q + _C, c], vals)
            return carry

        lax.fori_loop(0, 12, sc_group, 0)

        s = lane
        q = (s * 21846) >> 16
        c = s - q * 3
        m = s < 9
        vals = plsc.load_gather(tab_v, [s + 609], mask=m)
        plsc.store_scatter(out_v, [q, c], vals, mask=m)

        pltpu.sync_copy(out_v, out_hbm.at[h])

    return _bias


def kernel(rel_bias, context_sq_bias, sq_context_bias, context_context_bias):
    packed = jnp.concatenate(
        [
            rel_bias.reshape(_H, 225),
            context_sq_bias.reshape(_H, 192),
            sq_context_bias.reshape(_H, 192),
            context_context_bias.reshape(_H, 9),
        ],
        axis=1,
    )
    return _bias_fn()(packed)[:, :_S, :_S]


# speedup vs baseline: 1.2532x; 1.0044x over previous
import functools

import jax
import jax.numpy as jnp
from jax import lax
from jax.experimental import pallas as pl
from jax.experimental.pallas import tpu as pltpu
from jax.experimental.pallas import tpu_sc as plsc

_H = 32
_C = 3
_S = 67


def _i32(x):
    return jnp.full((16,), x, jnp.int32)


@functools.cache
def _bias_fn():
    mesh = plsc.VectorSubcoreMesh(core_axis_name="c", subcore_axis_name="s")

    @functools.partial(
        pl.kernel,
        out_type=jax.ShapeDtypeStruct((_H, 72, 128), jnp.float32),
        mesh=mesh,
        scratch_types=[
            pltpu.VMEM((8, 128), jnp.float32),
            pltpu.VMEM((72, 128), jnp.float32),
            pltpu.SemaphoreType.DMA,
        ],
        compiler_params=pltpu.CompilerParams(
            needs_layout_passes=False,
            use_tc_tiling_on_sc=False,
            disable_bounds_checks=True,
            disable_semaphore_checks=True,
            skip_device_barrier=True,
        ),
    )
    def _bias(tab_hbm, out_hbm, tab_v, out_v, sem0):
        num_cores = lax.axis_size("c")
        h = lax.axis_index("s") * num_cores + lax.axis_index("c")
        cp_tab = pltpu.make_async_copy(tab_hbm.at[h], tab_v, sem0)
        cp_tab.start()

        lane = lax.iota(jnp.int32, 16)
        cp_tab.wait()

        def sq_group(t, carry):
            i = t >> 2
            j = lane + ((t & 3) << 4)
            dr = ((i >> 3) + 7) - (j >> 3)
            df = ((i & 7) + 7) - (j & 7)
            f = dr * 15 + df
            vals = plsc.load_gather(tab_v, [f >> 7, f & 127])
            plsc.store_scatter(out_v, [_i32(i + _C), j + _C], vals)
            return carry

        lax.fori_loop(0, 256, sq_group, 0)

        def cs_group(t, carry):
            r = t >> 2
            c = lane + ((t & 3) << 4)
            f = (c + 225) + (r << 6)
            vals = plsc.load_gather(tab_v, [f >> 7, f & 127])
            plsc.store_scatter(out_v, [_i32(r), c + _C], vals)
            return carry

        lax.fori_loop(0, 12, cs_group, 0)

        def sc_group(v, carry):
            s = lane + (v << 4)
            q = (s * 21846) >> 16
            c = s - q * 3
            f = s + 417
            vals = plsc.load_gather(tab_v, [f >> 7, f & 127])
            plsc.store_scatter(out_v, [q + _C, c], vals)
            return carry

        lax.fori_loop(0, 12, sc_group, 0)

        s = lane
        q = (s * 21846) >> 16
        c = s - q * 3
        m = s < 9
        f = s + 609
        vals = plsc.load_gather(tab_v, [f >> 7, f & 127], mask=m)
        plsc.store_scatter(out_v, [q, c], vals, mask=m)

        pltpu.sync_copy(out_v, out_hbm.at[h])

    return _bias


def kernel(rel_bias, context_sq_bias, sq_context_bias, context_context_bias):
    packed = jnp.concatenate(
        [
            rel_bias.reshape(_H, 225),
            context_sq_bias.reshape(_H, 192),
            sq_context_bias.reshape(_H, 192),
            context_context_bias.reshape(_H, 9),
        ],
        axis=1,
    )
    packed = jnp.pad(packed, ((0, 0), (0, 406))).reshape(_H, 8, 128)
    return _bias_fn()(packed)[:, :_S, :_S]
